# Initial kernel scaffold; baseline (speedup 1.0000x reference)
import jax
import jax.numpy as jnp
from jax.experimental import pallas as pl


def kernel(tcr_idx, pos_donor_indices, donor_hla_matrix, binder_sets, z_table):
    raise NotImplementedError("write your pallas kernel here")


# speedup vs baseline: 1.7649x; 1.7649x over previous
import functools

import jax
import jax.numpy as jnp
from jax import lax
from jax.experimental import pallas as pl
from jax.experimental.pallas import tpu as pltpu
from jax.experimental.pallas import tpu_sc as plsc

NUM_TCRS = 100000
MAX_HLAS = 16
NUM_DONORS = 1024
NUM_HLAS = 200
BATCH = 1024
NUM_POS = 8
BETA = 4.0
L2_LAMBDA = 1e-05

_C_PAD = 256
_NC, _NS = 2, 16
_NW = _NC * _NS
_BPW = BATCH // _NW

_HALF_LOG_2PI = 0.9189385332046727


def _lgamma(x):
    y = x + 8.0
    yi = 1.0 / y
    yi2 = yi * yi
    s = (y - 0.5) * jnp.log(y) - y + _HALF_LOG_2PI
    s = s + yi * (8.333333333333333e-2
                  - yi2 * (2.777777777777778e-3 - yi2 * 7.936507936507937e-4))
    prod = (x * (x + 1.0) * (x + 2.0) * (x + 3.0)
            * (x + 4.0) * (x + 5.0) * (x + 6.0) * (x + 7.0))
    return s - jnp.log(prod)


@functools.cache
def _make_sc_gather():
    mesh = plsc.VectorSubcoreMesh(
        core_axis_name="c", subcore_axis_name="s",
        num_cores=_NC, num_subcores=_NS)

    @functools.partial(
        pl.kernel,
        out_type=(jax.ShapeDtypeStruct((BATCH, MAX_HLAS), jnp.float32),
                  jax.ShapeDtypeStruct((BATCH, MAX_HLAS), jnp.int32)),
        mesh=mesh,
        scratch_types=(
            pltpu.VMEM((_BPW,), jnp.int32),
            pltpu.VMEM((_BPW, MAX_HLAS), jnp.float32),
            pltpu.VMEM((_BPW, MAX_HLAS), jnp.int32),
            pltpu.SemaphoreType.DMA,
            pltpu.SemaphoreType.DMA,
        ),
        compiler_params=pltpu.CompilerParams(use_tc_tiling_on_sc=False),
    )
    def _sc_gather(idx_hbm, z_hbm, bind_hbm, z_out, b_out,
                   idx_v, z_v, b_v, sem_z, sem_b):
        wid = lax.axis_index("s") * _NC + lax.axis_index("c")
        base = wid * _BPW
        pltpu.sync_copy(idx_hbm.at[pl.ds(base, _BPW)], idx_v)
        cz = pltpu.async_copy(z_hbm.at[idx_v], z_v, sem_z)
        cb = pltpu.async_copy(bind_hbm.at[idx_v], b_v, sem_b)
        cz.wait()
        cb.wait()
        pltpu.sync_copy(z_v, z_out.at[pl.ds(base, _BPW)])
        pltpu.sync_copy(b_v, b_out.at[pl.ds(base, _BPW)])

    return _sc_gather


def _tc_body(z_ref, b_ref, xp_ref, pos_ref, nll_ref, reg_ref):
    z = z_ref[...]
    bidx = b_ref[...]
    pos = pos_ref[...]

    m = (bidx != -1).astype(jnp.float32)
    zp = m / (1.0 + jnp.exp(-z))
    w = jnp.log(jnp.maximum(1.0 - zp, 1e-7))

    ccol = lax.broadcasted_iota(jnp.int32, (BATCH, _C_PAD), 1)
    s = jnp.zeros((BATCH, _C_PAD), jnp.float32)
    for h in range(MAX_HLAS):
        s = s + jnp.where(ccol == bidx[:, h:h + 1], w[:, h:h + 1], 0.0)

    lp = jnp.dot(s, xp_ref[...], preferred_element_type=jnp.float32)
    p = jnp.maximum(1.0 - jnp.exp(lp), 1e-7)
    sum_p_all = jnp.sum(p, axis=1, keepdims=True)

    dcol = lax.broadcasted_iota(jnp.int32, (BATCH, NUM_DONORS), 1)
    reward = jnp.zeros((BATCH, 1), jnp.float32)
    sum_p_pos = jnp.zeros((BATCH, 1), jnp.float32)
    n_i = jnp.zeros((BATCH, 1), jnp.float32)
    for j in range(NUM_POS):
        pj = pos[:, j:j + 1]
        mj = (pj != -1).astype(jnp.float32)
        sel = jnp.where(dcol == jnp.maximum(pj, 0), p, 0.0)
        ppj = jnp.sum(sel, axis=1, keepdims=True)
        reward = reward + jnp.log(ppj) * mj
        sum_p_pos = sum_p_pos + ppj * mj
        n_i = n_i + mj

    n_tilde = sum_p_all - sum_p_pos
    pen = _lgamma(n_tilde + BETA) - _lgamma(n_i + n_tilde + BETA + 1.0)
    nll = -jnp.sum(reward + pen)
    n_valid = jnp.maximum(jnp.sum(m), 1.0)
    reg = L2_LAMBDA * jnp.sum(z * z * m) / n_valid
    nll_ref[...] = jnp.reshape(nll, (1, 1))
    reg_ref[...] = jnp.reshape(reg, (1, 1))


def _tc_compute(zrows, brows, xp, pos):
    return pl.pallas_call(
        _tc_body,
        out_shape=(jax.ShapeDtypeStruct((1, 1), jnp.float32),
                   jax.ShapeDtypeStruct((1, 1), jnp.float32)),
    )(zrows, brows, xp, pos)


def kernel(tcr_idx, pos_donor_indices, donor_hla_matrix, binder_sets, z_table):
    zrows, brows = _make_sc_gather()(tcr_idx, z_table, binder_sets)
    xp = jnp.pad(donor_hla_matrix.T, ((0, _C_PAD - NUM_HLAS), (0, 0)))
    nll, reg = _tc_compute(zrows, brows, xp, pos_donor_indices)
    return (nll[0, 0], reg[0, 0])
